# Initial kernel scaffold; baseline (speedup 1.0000x reference)
import jax
import jax.numpy as jnp
from jax.experimental import pallas as pl


def kernel(x, edge_index, conv1_w_rel, conv1_b_rel, conv1_w_root, conv2_w_rel, conv2_b_rel, conv2_w_root, fc1_w, fc1_b, fc2_w, fc2_b):
    raise NotImplementedError("write your pallas kernel here")


# speedup vs baseline: 1.0312x; 1.0312x over previous
import jax
import jax.numpy as jnp
from jax.experimental import pallas as pl


def _mlp_head_kernel(x_ref, w1_ref, b1_ref, w2_ref, b2_ref, o_ref):
    xf = x_ref[...]
    h = jax.lax.dot_general(
        xf, w1_ref[...],
        dimension_numbers=(((1,), (1,)), ((), ())),
        preferred_element_type=jnp.float32,
    )
    h = jnp.maximum(h + b1_ref[...], 0.0)
    o_ref[...] = jnp.sum(h * w2_ref[...], keepdims=True) + b2_ref[...]


def kernel(x, edge_index, conv1_w_rel, conv1_b_rel, conv1_w_root,
           conv2_w_rel, conv2_b_rel, conv2_w_root,
           fc1_w, fc1_b, fc2_w, fc2_b):
    out = pl.pallas_call(
        _mlp_head_kernel,
        out_shape=jax.ShapeDtypeStruct((1, 1), jnp.float32),
    )(x.reshape(1, 288), fc1_w, fc1_b.reshape(1, 288), fc2_w,
      fc2_b.reshape(1, 1))
    return out.reshape(1)
